# baseline (speedup 1.0000x reference)
import numpy as np
import jax
import jax.numpy as jnp
from jax.experimental import pallas as pl
from jax.experimental.pallas import tpu as pltpu

_CW = 2048
_SUP = 400


def _pick_block(n):
    for bm in (200, 128, 80, 40, 16, 8):
        if n % bm == 0:
            return bm
    return 1


def _served(r, sup, cw, nc):
    return sum(1 for c in range(nc - 1) if cw * (c + 1) <= sup * r)


def _make_body(nb, bsz, n, nc, lw, nrs, nres, cw, sup):
    nmain = nb
    spr = sup // bsz

    def finalize(rows_r, x_ref, t1_ref, t2a_ref, w0_ref, w1_ref, w2_ref,
                 b_ref, out_ref):
        out_ref[...] = (
            jnp.dot(x_ref[rows_r, :], w0_ref[...] - w2_ref[...],
                    preferred_element_type=jnp.float32)
            + jnp.dot(t1_ref[rows_r, :], w1_ref[...],
                      preferred_element_type=jnp.float32)
            + jnp.dot(t2a_ref[rows_r, :], 2.0 * w2_ref[...],
                      preferred_element_type=jnp.float32)
            + b_ref[...]
        )

    def body(ai, bi, bc, oo, x_ref, adja_ref, adjb_ref,
             w0_ref, w1_ref, w2_ref, b_ref,
             out_ref, t1_ref, t2a_ref):
        g = pl.program_id(0)

        @pl.when(g < nmain)
        def _main_sweep():
            i = g
            rows_i = pl.ds(i * bsz, bsz)
            stripe = adja_ref[...]
            t1c = jnp.dot(stripe, x_ref[...],
                          preferred_element_type=jnp.float32)
            t1_ref[rows_i, :] = t1c
            t2a_ref[rows_i, :] = jnp.zeros_like(t1c)
            served_rows = (bsz * i // sup) * sup
            for c in range(nc - 1):
                @pl.when(cw * (c + 1) <= served_rows)
                def _dual_serve(c=c):
                    t2a_ref[rows_i, :] += jnp.dot(
                        stripe[:, cw * c:cw * (c + 1)],
                        t1_ref[cw * c:cw * (c + 1), :],
                        precision=jax.lax.Precision.DEFAULT,
                        preferred_element_type=jnp.float32)

        @pl.when(jnp.logical_and(g >= nmain, g < nmain + nrs))
        def _restripe_sweep():
            s = ai[g]
            rows_s = pl.ds(s * bsz, bsz)
            t2a_ref[rows_s, :] = jnp.dot(
                adja_ref[...], t1_ref[...],
                precision=jax.lax.Precision.DEFAULT,
                preferred_element_type=jnp.float32)

            @pl.when(s % spr == spr - 1)
            def _finalize_super():
                finalize(pl.ds((s // spr) * sup, sup), x_ref, t1_ref,
                         t2a_ref, w0_ref, w1_ref, w2_ref, b_ref, out_ref)

        @pl.when(jnp.logical_and(g >= nmain + nrs, g < nmain + nrs + nres))
        def _residual_sweep():
            r = bi[g]
            c = bc[g]
            rows_r = pl.ds(r * sup, sup)
            t2a_ref[rows_r, :] += jnp.dot(
                adjb_ref[...], t1_ref[pl.ds(c * cw, cw), :],
                precision=jax.lax.Precision.DEFAULT,
                preferred_element_type=jnp.float32)

        @pl.when(g >= nmain + nrs + nres)
        def _last_chunk_and_finalize():
            r = bi[g]
            rows_r = pl.ds(r * sup, sup)
            t2a_ref[rows_r, :] += jnp.dot(
                adjb_ref[:, :lw], t1_ref[(nc - 1) * cw:(nc - 1) * cw + lw, :],
                precision=jax.lax.Precision.DEFAULT,
                preferred_element_type=jnp.float32)
            finalize(rows_r, x_ref, t1_ref, t2a_ref, w0_ref, w1_ref,
                     w2_ref, b_ref, out_ref)

    return body


def kernel(x, adj, W0, W1, W2, b):
    n, d_in = x.shape
    d_out = W0.shape[1]
    bsz = _pick_block(n)
    nb = n // bsz
    cw = min(_CW, n)
    sup = _SUP if (n % _SUP == 0 and _SUP % bsz == 0) else bsz
    nsup = n // sup
    spr = sup // bsz
    nc = -(-n // cw)
    lw = n - (nc - 1) * cw
    b2d = b.reshape(1, d_out).astype(jnp.float32)

    empty = [r for r in range(nsup) if _served(r, sup, cw, nc) == 0]
    nonempty = [r for r in range(nsup) if r not in empty]

    ai, bi, bc, oo = [], [], [], []
    for i in range(nb):
        ai.append(i)
        bi.append(0)
        bc.append(0)
        oo.append(0)
    for r in empty:
        for k in range(spr):
            ai.append(r * spr + k)
            bi.append(0)
            bc.append(0)
            oo.append(r)
    nrs = len(ai) - nb
    park_a = ai[-1] if ai else 0
    park_o = oo[-1] if oo else 0
    for r in nonempty:
        for c in range(nc - 1):
            if cw * (c + 1) > sup * r:
                ai.append(park_a)
                bi.append(r)
                bc.append(c)
                oo.append(park_o)
    nres = len(ai) - nb - nrs
    for r in nonempty:
        ai.append(park_a)
        bi.append(r)
        bc.append(nc - 1)
        oo.append(r)
    ai = jnp.asarray(np.array(ai, dtype=np.int32))
    bi = jnp.asarray(np.array(bi, dtype=np.int32))
    bc = jnp.asarray(np.array(bc, dtype=np.int32))
    oo = jnp.asarray(np.array(oo, dtype=np.int32))

    grid_spec = pltpu.PrefetchScalarGridSpec(
        num_scalar_prefetch=4,
        grid=(nb + nrs + nres + len(nonempty),),
        in_specs=[
            pl.BlockSpec((n, d_in), lambda g, a, i2, c2, o: (0, 0)),
            pl.BlockSpec((bsz, n), lambda g, a, i2, c2, o: (a[g], 0)),
            pl.BlockSpec((sup, cw), lambda g, a, i2, c2, o: (i2[g], c2[g])),
            pl.BlockSpec((d_in, d_out), lambda g, a, i2, c2, o: (0, 0)),
            pl.BlockSpec((d_in, d_out), lambda g, a, i2, c2, o: (0, 0)),
            pl.BlockSpec((d_in, d_out), lambda g, a, i2, c2, o: (0, 0)),
            pl.BlockSpec((1, d_out), lambda g, a, i2, c2, o: (0, 0)),
        ],
        out_specs=pl.BlockSpec((sup, d_out), lambda g, a, i2, c2, o: (o[g], 0)),
        scratch_shapes=[
            pltpu.VMEM((n, d_in), jnp.float32),
            pltpu.VMEM((n, d_out), jnp.float32),
        ],
    )
    out = pl.pallas_call(
        _make_body(nb, bsz, n, nc, lw, nrs, nres, cw, sup),
        grid_spec=grid_spec,
        out_shape=jax.ShapeDtypeStruct((n, d_out), jnp.float32),
        compiler_params=pltpu.CompilerParams(
            dimension_semantics=("arbitrary",),
            vmem_limit_bytes=100 * 1024 * 1024,
        ),
    )(ai, bi, bc, oo, x, adj, adj, W0, W1, W2, b2d)
    return out


# speedup vs baseline: 1.0633x; 1.0633x over previous
import numpy as np
import jax
import jax.numpy as jnp
from jax.experimental import pallas as pl
from jax.experimental.pallas import tpu as pltpu

_CW = 2048
_SUP = 1000


def _pick_block(n):
    for bm in (200, 128, 80, 40, 16, 8):
        if n % bm == 0:
            return bm
    return 1


def _make_body(nb, bsz, n, nc, lw, nres, cw, sup):
    nmain = nb

    def finalize(rows_r, x_ref, t1_ref, t2a_ref, w0_ref, w1_ref, w2_ref,
                 b_ref, out_ref):
        out_ref[...] = (
            jnp.dot(x_ref[rows_r, :], w0_ref[...] - w2_ref[...],
                    preferred_element_type=jnp.float32)
            + jnp.dot(t1_ref[rows_r, :], w1_ref[...],
                      preferred_element_type=jnp.float32)
            + jnp.dot(t2a_ref[rows_r, :], 2.0 * w2_ref[...],
                      preferred_element_type=jnp.float32)
            + b_ref[...]
        )

    def body(ai, bi, bc, oo, x_ref, adja_ref, adjb_ref,
             w0_ref, w1_ref, w2_ref, b_ref,
             out_ref, t1_ref, t2a_ref):
        g = pl.program_id(0)

        @pl.when(g < nmain)
        def _main_sweep():
            i = g
            rows_i = pl.ds(i * bsz, bsz)
            stripe = adja_ref[...]
            t1c = jnp.dot(stripe, x_ref[...],
                          preferred_element_type=jnp.float32)
            t1_ref[rows_i, :] = t1c
            t2a_ref[rows_i, :] = jnp.zeros_like(t1c)
            served_rows = (bsz * i // sup) * sup
            for c in range(nc - 1):
                @pl.when(cw * (c + 1) <= served_rows)
                def _dual_serve(c=c):
                    t2a_ref[rows_i, :] += jnp.dot(
                        stripe[:, cw * c:cw * (c + 1)],
                        t1_ref[cw * c:cw * (c + 1), :],
                        preferred_element_type=jnp.float32)

        @pl.when(jnp.logical_and(g >= nmain, g < nmain + nres))
        def _residual_sweep():
            r = bi[g]
            c = bc[g]
            rows_r = pl.ds(r * sup, sup)
            t2a_ref[rows_r, :] += jnp.dot(
                adjb_ref[...], t1_ref[pl.ds(c * cw, cw), :],
                preferred_element_type=jnp.float32)

        @pl.when(g >= nmain + nres)
        def _last_chunk_and_finalize():
            r = bi[g]
            rows_r = pl.ds(r * sup, sup)
            t2a_ref[rows_r, :] += jnp.dot(
                adjb_ref[:, :lw], t1_ref[(nc - 1) * cw:(nc - 1) * cw + lw, :],
                preferred_element_type=jnp.float32)
            finalize(rows_r, x_ref, t1_ref, t2a_ref, w0_ref, w1_ref,
                     w2_ref, b_ref, out_ref)

    return body


def kernel(x, adj, W0, W1, W2, b):
    n, d_in = x.shape
    d_out = W0.shape[1]
    bsz = _pick_block(n)
    nb = n // bsz
    cw = min(_CW, n)
    sup = _SUP if (n % _SUP == 0 and _SUP % bsz == 0) else bsz
    nsup = n // sup
    nc = -(-n // cw)
    lw = n - (nc - 1) * cw
    b2d = b.reshape(1, d_out).astype(jnp.float32)

    ai, bi, bc, oo = [], [], [], []
    for i in range(nb):
        ai.append(i)
        bi.append(0)
        bc.append(0)
        oo.append(0)
    for r in range(nsup):
        for c in range(nc - 1):
            if cw * (c + 1) > sup * r:
                ai.append(nb - 1)
                bi.append(r)
                bc.append(c)
                oo.append(0)
    nres = len(ai) - nb
    for r in range(nsup):
        ai.append(nb - 1)
        bi.append(r)
        bc.append(nc - 1)
        oo.append(r)
    ai = jnp.asarray(np.array(ai, dtype=np.int32))
    bi = jnp.asarray(np.array(bi, dtype=np.int32))
    bc = jnp.asarray(np.array(bc, dtype=np.int32))
    oo = jnp.asarray(np.array(oo, dtype=np.int32))

    grid_spec = pltpu.PrefetchScalarGridSpec(
        num_scalar_prefetch=4,
        grid=(nb + nres + nsup,),
        in_specs=[
            pl.BlockSpec((n, d_in), lambda g, a, i2, c2, o: (0, 0)),
            pl.BlockSpec((bsz, n), lambda g, a, i2, c2, o: (a[g], 0)),
            pl.BlockSpec((sup, cw), lambda g, a, i2, c2, o: (i2[g], c2[g])),
            pl.BlockSpec((d_in, d_out), lambda g, a, i2, c2, o: (0, 0)),
            pl.BlockSpec((d_in, d_out), lambda g, a, i2, c2, o: (0, 0)),
            pl.BlockSpec((d_in, d_out), lambda g, a, i2, c2, o: (0, 0)),
            pl.BlockSpec((1, d_out), lambda g, a, i2, c2, o: (0, 0)),
        ],
        out_specs=pl.BlockSpec((sup, d_out), lambda g, a, i2, c2, o: (o[g], 0)),
        scratch_shapes=[
            pltpu.VMEM((n, d_in), jnp.float32),
            pltpu.VMEM((n, d_out), jnp.float32),
        ],
    )
    out = pl.pallas_call(
        _make_body(nb, bsz, n, nc, lw, nres, cw, sup),
        grid_spec=grid_spec,
        out_shape=jax.ShapeDtypeStruct((n, d_out), jnp.float32),
        compiler_params=pltpu.CompilerParams(
            dimension_semantics=("arbitrary",),
            vmem_limit_bytes=100 * 1024 * 1024,
        ),
    )(ai, bi, bc, oo, x, adj, adj, W0, W1, W2, b2d)
    return out
